# Initial kernel scaffold; baseline (speedup 1.0000x reference)
import jax
import jax.numpy as jnp
from jax.experimental import pallas as pl


def kernel(x, emb_table, fc_w, fc_b):
    raise NotImplementedError("write your pallas kernel here")


# speedup vs baseline: 1.5346x; 1.5346x over previous
import functools

import jax
import jax.numpy as jnp
from jax import lax
from jax.experimental import pallas as pl
from jax.experimental.pallas import tpu as pltpu
from jax.experimental.pallas import tpu_sc as plsc

VOCAB = 1000000
EMB = 32
N_LABELS = 100
B, L = 4096, 200

NC, NS = 2, 16
NW = NC * NS
ROWS_PER_W = B // NW
LH = 104
HALVES_PER_W = 2 * ROWS_PER_W


def _pool_body(table_hbm, idx_hbm, out_hbm, idx_v, buf0, buf1, sums_v, sem):
    wid = lax.axis_index("s") * NC + lax.axis_index("c")
    hbase = wid * HALVES_PER_W
    rbase = wid * ROWS_PER_W
    pltpu.sync_copy(idx_hbm.at[pl.ds(hbase, HALVES_PER_W), :], idx_v)

    def row(r, _):
        r2 = r * 2
        cp0 = pltpu.async_copy(table_hbm.at[idx_v.at[r2]], buf0, sem)
        cp1 = pltpu.async_copy(table_hbm.at[idx_v.at[r2 + 1]], buf1, sem)
        cp0.wait()
        cp1.wait()
        z = jnp.zeros((16,), jnp.float32)
        a0 = a1 = b0 = b1 = c0 = c1 = d0 = d1 = z
        for l in range(0, LH, 2):
            a0 = a0 + buf0[l, pl.ds(0, 16)]
            a1 = a1 + buf0[l, pl.ds(16, 16)]
            b0 = b0 + buf0[l + 1, pl.ds(0, 16)]
            b1 = b1 + buf0[l + 1, pl.ds(16, 16)]
            c0 = c0 + buf1[l, pl.ds(0, 16)]
            c1 = c1 + buf1[l, pl.ds(16, 16)]
            d0 = d0 + buf1[l + 1, pl.ds(0, 16)]
            d1 = d1 + buf1[l + 1, pl.ds(16, 16)]
        sums_v[r, pl.ds(0, 16)] = (a0 + b0) + (c0 + d0)
        sums_v[r, pl.ds(16, 16)] = (a1 + b1) + (c1 + d1)
        return 0

    lax.fori_loop(0, ROWS_PER_W, row, 0)
    pltpu.sync_copy(sums_v, out_hbm.at[pl.ds(rbase, ROWS_PER_W), :])


_pool = functools.partial(
    pl.kernel,
    mesh=plsc.VectorSubcoreMesh(core_axis_name="c", subcore_axis_name="s"),
    out_type=jax.ShapeDtypeStruct((B, EMB), jnp.float32),
    scratch_types=[
        pltpu.VMEM((HALVES_PER_W, LH), jnp.int32),
        pltpu.VMEM((LH, EMB), jnp.float32),
        pltpu.VMEM((LH, EMB), jnp.float32),
        pltpu.VMEM((ROWS_PER_W, EMB), jnp.float32),
        pltpu.SemaphoreType.DMA,
    ],
    compiler_params=pltpu.CompilerParams(use_tc_tiling_on_sc=False),
)(_pool_body)


def _head_body(x_ref, sums_ref, w_ref, b_ref, out_ref):
    mask = (x_ref[...] != 0).astype(jnp.float32)
    cnt = jnp.maximum(jnp.sum(mask, axis=1, keepdims=True), 1.0)
    avg = sums_ref[...] / cnt
    out_ref[...] = lax.dot_general(
        avg, w_ref[...], (((1,), (1,)), ((), ())),
        preferred_element_type=jnp.float32,
    ) + b_ref[...]


_HEAD_BLK = 512


def _head(x, sums, fc_w, fc_b2):
    return pl.pallas_call(
        _head_body,
        grid=(B // _HEAD_BLK,),
        in_specs=[
            pl.BlockSpec((_HEAD_BLK, L), lambda i: (i, 0)),
            pl.BlockSpec((_HEAD_BLK, EMB), lambda i: (i, 0)),
            pl.BlockSpec((N_LABELS, EMB), lambda i: (0, 0)),
            pl.BlockSpec((1, N_LABELS), lambda i: (0, 0)),
        ],
        out_specs=pl.BlockSpec((_HEAD_BLK, N_LABELS), lambda i: (i, 0)),
        out_shape=jax.ShapeDtypeStruct((B, N_LABELS), jnp.float32),
    )(x, sums, fc_w, fc_b2)


@jax.jit
def kernel(x, emb_table, fc_w, fc_b):
    xp = jnp.pad(x, ((0, 0), (0, 2 * LH - L))).reshape(2 * B, LH)
    sums = _pool(emb_table, xp)
    return _head(x, sums, fc_w, fc_b.reshape(1, N_LABELS))


# speedup vs baseline: 2.1886x; 1.4262x over previous
import functools

import jax
import jax.numpy as jnp
from jax import lax
from jax.experimental import pallas as pl
from jax.experimental.pallas import tpu as pltpu
from jax.experimental.pallas import tpu_sc as plsc

VOCAB = 1000000
EMB = 32
N_LABELS = 100
B, L = 4096, 200

NC, NS = 2, 16
NW = NC * NS
ROWS_PER_W = B // NW
LH = 104
NPAD = 2 * LH - L
HALVES_PER_W = 2 * ROWS_PER_W


def _accum_row(buf0, buf1, sums_v, r):
    z = jnp.zeros((16,), jnp.float32)
    a0 = a1 = b0 = b1 = c0 = c1 = d0 = d1 = z
    for l in range(0, LH, 2):
        a0 = a0 + buf0[l, pl.ds(0, 16)]
        a1 = a1 + buf0[l, pl.ds(16, 16)]
        b0 = b0 + buf0[l + 1, pl.ds(0, 16)]
        b1 = b1 + buf0[l + 1, pl.ds(16, 16)]
        if l + 1 < LH - NPAD:
            c0 = c0 + buf1[l, pl.ds(0, 16)]
            c1 = c1 + buf1[l, pl.ds(16, 16)]
            d0 = d0 + buf1[l + 1, pl.ds(0, 16)]
            d1 = d1 + buf1[l + 1, pl.ds(16, 16)]
    sums_v[r, pl.ds(0, 16)] = (a0 + b0) + (c0 + d0)
    sums_v[r, pl.ds(16, 16)] = (a1 + b1) + (c1 + d1)


def _pool_body(table_hbm, idx_hbm, out_hbm, idx_v,
               a0_v, a1_v, b0_v, b1_v, sums_v, sem_a, sem_b):
    wid = lax.axis_index("s") * NC + lax.axis_index("c")
    hbase = wid * HALVES_PER_W
    rbase = wid * ROWS_PER_W
    pltpu.sync_copy(idx_hbm.at[pl.ds(hbase, HALVES_PER_W), :], idx_v)

    def fire(rr, b0, b1, sem):
        pltpu.async_copy(table_hbm.at[idx_v.at[rr]], b0, sem)
        pltpu.async_copy(table_hbm.at[idx_v.at[rr + 1]], b1, sem)

    def drain(b0, b1, sem):
        pltpu.make_async_copy(table_hbm.at[pl.ds(0, LH)], b0, sem).wait()
        pltpu.make_async_copy(table_hbm.at[pl.ds(0, LH)], b1, sem).wait()

    fire(0, a0_v, a1_v, sem_a)

    def pair(p, _):
        fire(4 * p + 2, b0_v, b1_v, sem_b)
        drain(a0_v, a1_v, sem_a)
        _accum_row(a0_v, a1_v, sums_v, 2 * p)

        @pl.when(p < ROWS_PER_W // 2 - 1)
        def _():
            fire(4 * p + 4, a0_v, a1_v, sem_a)

        drain(b0_v, b1_v, sem_b)
        _accum_row(b0_v, b1_v, sums_v, 2 * p + 1)
        return 0

    lax.fori_loop(0, ROWS_PER_W // 2, pair, 0)
    pltpu.sync_copy(sums_v, out_hbm.at[pl.ds(rbase, ROWS_PER_W), :])


_pool = functools.partial(
    pl.kernel,
    mesh=plsc.VectorSubcoreMesh(core_axis_name="c", subcore_axis_name="s"),
    out_type=jax.ShapeDtypeStruct((B, EMB), jnp.float32),
    scratch_types=[
        pltpu.VMEM((HALVES_PER_W, LH), jnp.int32),
        pltpu.VMEM((LH, EMB), jnp.float32),
        pltpu.VMEM((LH, EMB), jnp.float32),
        pltpu.VMEM((LH, EMB), jnp.float32),
        pltpu.VMEM((LH, EMB), jnp.float32),
        pltpu.VMEM((ROWS_PER_W, EMB), jnp.float32),
        pltpu.SemaphoreType.DMA,
        pltpu.SemaphoreType.DMA,
    ],
    compiler_params=pltpu.CompilerParams(use_tc_tiling_on_sc=False),
)(_pool_body)


def _head_body(x_ref, sums_ref, w_ref, b_ref, out_ref):
    mask = (x_ref[...] != 0).astype(jnp.float32)
    cnt = jnp.maximum(jnp.sum(mask, axis=1, keepdims=True), 1.0)
    avg = sums_ref[...] / cnt
    out_ref[...] = lax.dot_general(
        avg, w_ref[...], (((1,), (1,)), ((), ())),
        preferred_element_type=jnp.float32,
    ) + b_ref[...]


_HEAD_BLK = 512


def _head(x, sums, fc_w, fc_b2):
    return pl.pallas_call(
        _head_body,
        grid=(B // _HEAD_BLK,),
        in_specs=[
            pl.BlockSpec((_HEAD_BLK, L), lambda i: (i, 0)),
            pl.BlockSpec((_HEAD_BLK, EMB), lambda i: (i, 0)),
            pl.BlockSpec((N_LABELS, EMB), lambda i: (0, 0)),
            pl.BlockSpec((1, N_LABELS), lambda i: (0, 0)),
        ],
        out_specs=pl.BlockSpec((_HEAD_BLK, N_LABELS), lambda i: (i, 0)),
        out_shape=jax.ShapeDtypeStruct((B, N_LABELS), jnp.float32),
    )(x, sums, fc_w, fc_b2)


@jax.jit
def kernel(x, emb_table, fc_w, fc_b):
    dummy = (jnp.arange(B * NPAD, dtype=jnp.int32) % VOCAB).reshape(B, NPAD)
    xp = jnp.concatenate([x, dummy], axis=1).reshape(2 * B, LH)
    sums = _pool(emb_table, xp)
    return _head(x, sums, fc_w, fc_b.reshape(1, N_LABELS))
